# speedup vs baseline: 32.5884x; 32.5884x over previous
import jax
import jax.numpy as jnp
from jax.experimental import pallas as pl
from jax.experimental.pallas import tpu as pltpu

N_CHUNKS = 8
_LN2 = 0.6931471805599453
_BIG = 2147483647


def _prep_body(p0, p1, p6, xx, yy, sex, sey, sm):
    sex[...] = jnp.tanh(p0[...]) + xx[...]
    sey[...] = jnp.tanh(p1[...]) + yy[...]
    sm[...] = jax.nn.sigmoid(p6[...])


def _cluster_body(sex, sey, sgx, sgy, sm, lab):
    H, W = sm.shape
    CH = H // N_CHUNKS
    f32 = jnp.float32
    i32 = jnp.int32

    def chunk_iota(i):
        r = jax.lax.broadcasted_iota(i32, (CH, W), 0) + i * CH
        c = jax.lax.broadcasted_iota(i32, (CH, W), 1)
        return r * W + c

    def init_chunk(i, carry):
        tot, mx, am = carry
        sl = pl.ds(i * CH, CH)
        s = sm[sl, :]
        v = s > 0.5
        lab[sl, :] = jnp.where(v, i32(0), i32(-1))
        sc = jnp.where(v, s, f32(0.0))
        tot = tot + jnp.sum(v.astype(f32))
        m = jnp.max(sc)
        cand = jnp.min(jnp.where(sc == m, chunk_iota(i), i32(_BIG)))
        take = m > mx
        return (tot, jnp.where(take, m, mx), jnp.where(take, cand, am))

    tot0, mx0, am0 = jax.lax.fori_loop(
        0, N_CHUNKS, init_chunk, (f32(0.0), f32(-1.0), i32(0)))

    def cond_fn(st):
        tot, mx, am, cnt = st
        return (tot > 160.0) & (mx >= 0.5)

    def body_fn(st):
        tot, mx, am, cnt = st
        r = am // W
        c = am % W
        sel = jax.lax.broadcasted_iota(i32, (1, W), 1) == c

        def pick(ref):
            return jnp.sum(jnp.where(sel, ref[pl.ds(r, 1), :], f32(0.0)))

        cx = pick(sex)
        cy = pick(sey)
        ssx = jnp.exp(pick(sgx) * 10.0)
        ssy = jnp.exp(pick(sgy) * 10.0)

        def prop_chunk(i):
            sl = pl.ds(i * CH, CH)
            q = (jnp.square(sex[sl, :] - cx) * ssx
                 + jnp.square(sey[sl, :] - cy) * ssy)
            l = lab[sl, :]
            return (q < _LN2) & (l != -1), l

        def pass_reduce(i, carry):
            ps, rn = carry
            prop, l = prop_chunk(i)
            ps = ps + jnp.sum(prop.astype(f32))
            rn = rn + jnp.sum((prop & (l == 0)).astype(f32))
            return (ps, rn)

        psum, rnum = jax.lax.fori_loop(
            0, N_CHUNKS, pass_reduce, (f32(0.0), f32(0.0)))
        rnum = rnum - 1.0
        add = (psum > 160.0) & (rnum / jnp.maximum(psum, 1.0) > 0.5)

        def pass_update(i, carry):
            tot, mx, am = carry
            sl = pl.ds(i * CH, CH)
            prop, l = prop_chunk(i)
            lnew = jnp.where(
                prop,
                jnp.where(add, cnt, jnp.where(l == 0, i32(-2), l)),
                l)
            lab[sl, :] = lnew
            un = lnew == 0
            sc = jnp.where(un, sm[sl, :], f32(0.0))
            tot = tot + jnp.sum(un.astype(f32))
            m = jnp.max(sc)
            cand = jnp.min(jnp.where(sc == m, chunk_iota(i), i32(_BIG)))
            take = m > mx
            return (tot, jnp.where(take, m, mx), jnp.where(take, cand, am))

        tot, mx, am = jax.lax.fori_loop(
            0, N_CHUNKS, pass_update, (f32(0.0), f32(-1.0), i32(0)))
        cnt = cnt + add.astype(i32)
        return (tot, mx, am, cnt)

    jax.lax.while_loop(cond_fn, body_fn, (tot0, mx0, am0, jnp.int32(1)))


def kernel(prediction, xym):
    pred = prediction[0]
    H, W = pred.shape[1], pred.shape[2]
    CH = H // N_CHUNKS
    xx = xym[0, 0:H, 0:W]
    yy = xym[1, 0:H, 0:W]

    blk = pl.BlockSpec((CH, W), lambda i: (i, 0))
    sex, sey, sm = pl.pallas_call(
        _prep_body,
        grid=(N_CHUNKS,),
        in_specs=[blk] * 5,
        out_specs=[blk] * 3,
        out_shape=[jax.ShapeDtypeStruct((H, W), jnp.float32)] * 3,
    )(pred[0], pred[1], pred[6], xx, yy)

    full = pl.BlockSpec(memory_space=pltpu.VMEM)
    lab = pl.pallas_call(
        _cluster_body,
        in_specs=[full] * 5,
        out_specs=full,
        out_shape=jax.ShapeDtypeStruct((H, W), jnp.int32),
    )(sex, sey, pred[2], pred[3], sm)

    inst = (jnp.maximum(lab, 0) % 256).astype(jnp.uint8)
    return inst[None]


# speedup vs baseline: 42.2666x; 1.2970x over previous
import jax
import jax.numpy as jnp
from jax.experimental import pallas as pl
from jax.experimental.pallas import tpu as pltpu

N_CHUNKS = 8
N_PREP = 16
_LN2 = 0.6931471805599453
_BIG = 2147483647


def _cluster_body(p0, p1, p6, xx, yy, sgx, sgy, out,
                  sex, sey, sm, lab, prop8, stage, rowx, rowy,
                  psem, rsem):
    H, W = sex.shape
    CH = H // N_CHUNKS
    CHP = H // N_PREP
    f32 = jnp.float32
    i32 = jnp.int32
    srcs = (p0, p1, p6, xx, yy)

    def chunk_iota(i, rows):
        r = jax.lax.broadcasted_iota(i32, (rows, W), 0) + i * rows
        c = jax.lax.broadcasted_iota(i32, (rows, W), 1)
        return r * W + c

    def stage_copy(slot, i, j):
        return pltpu.make_async_copy(
            srcs[j].at[pl.ds(i * CHP, CHP), :], stage.at[slot, j],
            psem.at[slot, j])

    def issue(slot, i):
        for j in range(5):
            stage_copy(slot, i, j).start()

    def wait_all(slot, i):
        for j in range(5):
            stage_copy(slot, i, j).wait()

    issue(0, 0)

    def prep_it(i, carry):
        tot, mx, am = carry
        slot = jax.lax.rem(i, 2)

        @pl.when(i + 1 < N_PREP)
        def _():
            issue(jax.lax.rem(i + 1, 2), i + 1)

        wait_all(slot, i)
        sl = pl.ds(i * CHP, CHP)
        sex[sl, :] = jnp.tanh(stage[slot, 0]) + stage[slot, 3]
        sey[sl, :] = jnp.tanh(stage[slot, 1]) + stage[slot, 4]
        s = jax.nn.sigmoid(stage[slot, 2])
        sm[sl, :] = s
        v = s > 0.5
        lab[sl, :] = jnp.where(v, i32(0), i32(-1))
        sc = jnp.where(v, s, f32(0.0))
        tot = tot + jnp.sum(v.astype(f32))
        m = jnp.max(sc)
        cand = jnp.min(jnp.where(sc == m, chunk_iota(i, CHP), i32(_BIG)))
        take = m > mx
        return (tot, jnp.where(take, m, mx), jnp.where(take, cand, am))

    tot0, mx0, am0 = jax.lax.fori_loop(
        0, N_PREP, prep_it, (f32(0.0), f32(-1.0), i32(0)))

    def cond_fn(st):
        tot, mx, am, cnt = st
        return (tot > 160.0) & (mx >= 0.5)

    def body_fn(st):
        tot, mx, am, cnt = st
        r = am // W
        c = am % W
        cpx = pltpu.make_async_copy(sgx.at[pl.ds(r, 1), :], rowx,
                                    rsem.at[0])
        cpy = pltpu.make_async_copy(sgy.at[pl.ds(r, 1), :], rowy,
                                    rsem.at[1])
        cpx.start()
        cpy.start()
        sel = jax.lax.broadcasted_iota(i32, (1, W), 1) == c

        def pick(ref):
            return jnp.sum(jnp.where(sel, ref[pl.ds(r, 1), :], f32(0.0)))

        def pick_row(ref):
            return jnp.sum(jnp.where(sel, ref[...], f32(0.0)))

        cx = pick(sex)
        cy = pick(sey)
        cpx.wait()
        cpy.wait()
        ssx = jnp.exp(pick_row(rowx) * 10.0)
        ssy = jnp.exp(pick_row(rowy) * 10.0)

        def pass_reduce(i, carry):
            ps, rn = carry
            sl = pl.ds(i * CH, CH)
            q = (jnp.square(sex[sl, :] - cx) * ssx
                 + jnp.square(sey[sl, :] - cy) * ssy)
            l = lab[sl, :]
            prop = (q < _LN2) & (l != -1)
            prop8[sl, :] = prop.astype(jnp.int8)
            ps = ps + jnp.sum(prop.astype(f32))
            rn = rn + jnp.sum((prop & (l == 0)).astype(f32))
            return (ps, rn)

        psum, rnum = jax.lax.fori_loop(
            0, N_CHUNKS, pass_reduce, (f32(0.0), f32(0.0)))
        rnum = rnum - 1.0
        add = (psum > 160.0) & (rnum / jnp.maximum(psum, 1.0) > 0.5)

        def pass_update(i, carry):
            tot, mx, am = carry
            sl = pl.ds(i * CH, CH)
            prop = prop8[sl, :] != 0
            l = lab[sl, :]
            lnew = jnp.where(
                prop,
                jnp.where(add, cnt, jnp.where(l == 0, i32(-2), l)),
                l)
            lab[sl, :] = lnew
            un = lnew == 0
            sc = jnp.where(un, sm[sl, :], f32(0.0))
            tot = tot + jnp.sum(un.astype(f32))
            m = jnp.max(sc)
            cand = jnp.min(jnp.where(sc == m, chunk_iota(i, CH), i32(_BIG)))
            take = m > mx
            return (tot, jnp.where(take, m, mx), jnp.where(take, cand, am))

        tot, mx, am = jax.lax.fori_loop(
            0, N_CHUNKS, pass_update, (f32(0.0), f32(-1.0), i32(0)))
        cnt = cnt + add.astype(i32)
        return (tot, mx, am, cnt)

    jax.lax.while_loop(cond_fn, body_fn, (tot0, mx0, am0, jnp.int32(1)))

    def emit(i, _):
        sl = pl.ds(i * CH, CH)
        l = jnp.maximum(lab[sl, :], 0)
        out[sl, :] = jnp.bitwise_and(l, 255).astype(jnp.uint8)
        return 0

    jax.lax.fori_loop(0, N_CHUNKS, emit, 0)


def kernel(prediction, xym):
    pred = prediction[0]
    H, W = pred.shape[1], pred.shape[2]
    CHP = H // N_PREP
    xx = xym[0, 0:H, 0:W]
    yy = xym[1, 0:H, 0:W]

    hbm = pl.BlockSpec(memory_space=pl.ANY)
    inst = pl.pallas_call(
        _cluster_body,
        in_specs=[hbm] * 7,
        out_specs=pl.BlockSpec(memory_space=pltpu.VMEM),
        out_shape=jax.ShapeDtypeStruct((H, W), jnp.uint8),
        scratch_shapes=[
            pltpu.VMEM((H, W), jnp.float32),
            pltpu.VMEM((H, W), jnp.float32),
            pltpu.VMEM((H, W), jnp.float32),
            pltpu.VMEM((H, W), jnp.int32),
            pltpu.VMEM((H, W), jnp.int8),
            pltpu.VMEM((2, 5, CHP, W), jnp.float32),
            pltpu.VMEM((1, W), jnp.float32),
            pltpu.VMEM((1, W), jnp.float32),
            pltpu.SemaphoreType.DMA((2, 5)),
            pltpu.SemaphoreType.DMA((2,)),
        ],
    )(pred[0], pred[1], pred[6], xx, yy, pred[2], pred[3])

    return inst[None]


# speedup vs baseline: 48.3958x; 1.1450x over previous
import jax
import jax.numpy as jnp
from jax.experimental import pallas as pl
from jax.experimental.pallas import tpu as pltpu

N_CHUNKS = 4
N_PREP = 16
_LN2 = 0.6931471805599453
_BIG = 2147483647


def _cluster_body(p0, p1, p6, xx, yy, sgx, sgy, out,
                  sex, sey, score, lab, prop8, stage, rowx, rowy,
                  psem, rsem):
    H, W = sex.shape
    CH = H // N_CHUNKS
    CHP = H // N_PREP
    f32 = jnp.float32
    i32 = jnp.int32
    INF = jnp.float32(jnp.inf)
    srcs = (p0, p1, p6, xx, yy)

    def chunk_iota(i, rows):
        r = jax.lax.broadcasted_iota(i32, (rows, W), 0) + i * rows
        c = jax.lax.broadcasted_iota(i32, (rows, W), 1)
        return r * W + c

    def stage_copy(slot, i, j):
        return pltpu.make_async_copy(
            srcs[j].at[pl.ds(i * CHP, CHP), :], stage.at[slot, j],
            psem.at[slot, j])

    def issue(slot, i):
        for j in range(5):
            stage_copy(slot, i, j).start()

    def wait_all(slot, i):
        for j in range(5):
            stage_copy(slot, i, j).wait()

    issue(0, 0)

    def prep_it(i, carry):
        tot, mx, am = carry
        slot = jax.lax.rem(i, 2)

        @pl.when(i + 1 < N_PREP)
        def _():
            issue(jax.lax.rem(i + 1, 2), i + 1)

        wait_all(slot, i)
        sl = pl.ds(i * CHP, CHP)
        s = jax.nn.sigmoid(stage[slot, 2])
        v = s > 0.5
        sex[sl, :] = jnp.where(v, jnp.tanh(stage[slot, 0]) + stage[slot, 3],
                               INF)
        sey[sl, :] = jnp.where(v, jnp.tanh(stage[slot, 1]) + stage[slot, 4],
                               INF)
        sc = jnp.where(v, s, f32(0.0))
        score[sl, :] = sc
        lab[sl, :] = jnp.where(v, i32(0), i32(-1))
        tot = tot + jnp.sum(v.astype(f32))
        m = jnp.max(sc)
        cand = jnp.min(jnp.where(sc == m, chunk_iota(i, CHP), i32(_BIG)))
        take = m > mx
        return (tot, jnp.where(take, m, mx), jnp.where(take, cand, am))

    tot0, mx0, am0 = jax.lax.fori_loop(
        0, N_PREP, prep_it, (f32(0.0), f32(-1.0), i32(0)))

    def cond_fn(st):
        tot, mx, am, cnt = st
        return (tot > 160.0) & (mx >= 0.5)

    def body_fn(st):
        tot, mx, am, cnt = st
        r = am // W
        c = am % W
        cpx = pltpu.make_async_copy(sgx.at[pl.ds(r, 1), :], rowx,
                                    rsem.at[0])
        cpy = pltpu.make_async_copy(sgy.at[pl.ds(r, 1), :], rowy,
                                    rsem.at[1])
        cpx.start()
        cpy.start()
        sel = jax.lax.broadcasted_iota(i32, (1, W), 1) == c

        def pick(ref):
            return jnp.sum(jnp.where(sel, ref[pl.ds(r, 1), :], f32(0.0)))

        def pick_row(ref):
            return jnp.sum(jnp.where(sel, ref[...], f32(0.0)))

        cx = pick(sex)
        cy = pick(sey)
        cpx.wait()
        cpy.wait()
        ssx = jnp.exp(pick_row(rowx) * 10.0)
        ssy = jnp.exp(pick_row(rowy) * 10.0)

        def fused(i, carry):
            ps, rn, tot, mx, am = carry
            sl = pl.ds(i * CH, CH)
            q = (jnp.square(sex[sl, :] - cx) * ssx
                 + jnp.square(sey[sl, :] - cy) * ssy)
            prop = q < _LN2
            prop8[sl, :] = prop.astype(jnp.int8)
            sc = score[sl, :]
            ps = ps + jnp.sum(prop.astype(f32))
            rn = rn + jnp.sum((prop & (sc > 0)).astype(f32))
            sc = jnp.where(prop, f32(0.0), sc)
            score[sl, :] = sc
            tot = tot + jnp.sum((sc > 0).astype(f32))
            m = jnp.max(sc)
            cand = jnp.min(jnp.where(sc == m, chunk_iota(i, CH), i32(_BIG)))
            take = m > mx
            return (ps, rn, tot, jnp.where(take, m, mx),
                    jnp.where(take, cand, am))

        psum, rnum, tot, mx, am = jax.lax.fori_loop(
            0, N_CHUNKS, fused,
            (f32(0.0), f32(0.0), f32(0.0), f32(-1.0), i32(0)))
        rnum = rnum - 1.0
        add = (psum > 160.0) & (rnum / jnp.maximum(psum, 1.0) > 0.5)

        @pl.when(add)
        def _():
            def assign(i, _):
                sl = pl.ds(i * CH, CH)
                pr = prop8[sl, :].astype(i32) != 0
                lab[sl, :] = jnp.where(pr, cnt, lab[sl, :])
                return 0

            jax.lax.fori_loop(0, N_CHUNKS, assign, 0)

        cnt = cnt + add.astype(i32)
        return (tot, mx, am, cnt)

    jax.lax.while_loop(cond_fn, body_fn, (tot0, mx0, am0, jnp.int32(1)))

    def emit(i, _):
        sl = pl.ds(i * CH, CH)
        l = jnp.maximum(lab[sl, :], 0)
        out[sl, :] = jnp.bitwise_and(l, 255).astype(jnp.uint8)
        return 0

    jax.lax.fori_loop(0, N_CHUNKS, emit, 0)


def kernel(prediction, xym):
    pred = prediction[0]
    H, W = pred.shape[1], pred.shape[2]
    CHP = H // N_PREP
    xx = xym[0, 0:H, 0:W]
    yy = xym[1, 0:H, 0:W]

    hbm = pl.BlockSpec(memory_space=pl.ANY)
    inst = pl.pallas_call(
        _cluster_body,
        in_specs=[hbm] * 7,
        out_specs=pl.BlockSpec(memory_space=pltpu.VMEM),
        out_shape=jax.ShapeDtypeStruct((H, W), jnp.uint8),
        scratch_shapes=[
            pltpu.VMEM((H, W), jnp.float32),
            pltpu.VMEM((H, W), jnp.float32),
            pltpu.VMEM((H, W), jnp.float32),
            pltpu.VMEM((H, W), jnp.int32),
            pltpu.VMEM((H, W), jnp.int8),
            pltpu.VMEM((2, 5, CHP, W), jnp.float32),
            pltpu.VMEM((1, W), jnp.float32),
            pltpu.VMEM((1, W), jnp.float32),
            pltpu.SemaphoreType.DMA((2, 5)),
            pltpu.SemaphoreType.DMA((2,)),
        ],
    )(pred[0], pred[1], pred[6], xx, yy, pred[2], pred[3])

    return inst[None]


# speedup vs baseline: 50.5722x; 1.0450x over previous
import jax
import jax.numpy as jnp
from jax.experimental import pallas as pl
from jax.experimental.pallas import tpu as pltpu

N_CHUNKS = 2
N_PREP = 16
_LN2 = 0.6931471805599453
_BIG = 2147483647


def _cluster_body(p0, p1, p6, xx, yy, sgx, sgy, out,
                  sex, sey, score, lab, prop8, stage, rowx, rowy,
                  psem, rsem):
    H, W = sex.shape
    CH = H // N_CHUNKS
    CHP = H // N_PREP
    f32 = jnp.float32
    i32 = jnp.int32
    INF = jnp.float32(jnp.inf)
    srcs = (p0, p1, p6, xx, yy)

    def chunk_iota(i, rows):
        r = jax.lax.broadcasted_iota(i32, (rows, W), 0) + i * rows
        c = jax.lax.broadcasted_iota(i32, (rows, W), 1)
        return r * W + c

    def stage_copy(slot, i, j):
        return pltpu.make_async_copy(
            srcs[j].at[pl.ds(i * CHP, CHP), :], stage.at[slot, j],
            psem.at[slot, j])

    def issue(slot, i):
        for j in range(5):
            stage_copy(slot, i, j).start()

    def wait_all(slot, i):
        for j in range(5):
            stage_copy(slot, i, j).wait()

    issue(0, 0)

    def prep_it(i, carry):
        tot, mx, am = carry
        slot = jax.lax.rem(i, 2)

        @pl.when(i + 1 < N_PREP)
        def _():
            issue(jax.lax.rem(i + 1, 2), i + 1)

        wait_all(slot, i)
        sl = pl.ds(i * CHP, CHP)
        s = jax.nn.sigmoid(stage[slot, 2])
        v = s > 0.5
        sex[sl, :] = jnp.where(v, jnp.tanh(stage[slot, 0]) + stage[slot, 3],
                               INF)
        sey[sl, :] = jnp.where(v, jnp.tanh(stage[slot, 1]) + stage[slot, 4],
                               INF)
        sc = jnp.where(v, s, f32(0.0))
        score[sl, :] = sc
        lab[sl, :] = jnp.where(v, i32(0), i32(-1))
        tot = tot + jnp.sum(v.astype(f32))
        m = jnp.max(sc)
        cand = jnp.min(jnp.where(sc == m, chunk_iota(i, CHP), i32(_BIG)))
        take = m > mx
        return (tot, jnp.where(take, m, mx), jnp.where(take, cand, am))

    tot0, mx0, am0 = jax.lax.fori_loop(
        0, N_PREP, prep_it, (f32(0.0), f32(-1.0), i32(0)))

    def cond_fn(st):
        tot, mx, am, cnt = st
        return (tot > 160.0) & (mx >= 0.5)

    def body_fn(st):
        tot, mx, am, cnt = st
        r = am // W
        c = am % W
        cpx = pltpu.make_async_copy(sgx.at[pl.ds(r, 1), :], rowx,
                                    rsem.at[0])
        cpy = pltpu.make_async_copy(sgy.at[pl.ds(r, 1), :], rowy,
                                    rsem.at[1])
        cpx.start()
        cpy.start()
        sel = jax.lax.broadcasted_iota(i32, (1, W), 1) == c

        def pick(ref):
            return jnp.sum(jnp.where(sel, ref[pl.ds(r, 1), :], f32(0.0)))

        def pick_row(ref):
            return jnp.sum(jnp.where(sel, ref[...], f32(0.0)))

        cx = pick(sex)
        cy = pick(sey)
        cpx.wait()
        cpy.wait()
        ssx = jnp.exp(pick_row(rowx) * 10.0)
        ssy = jnp.exp(pick_row(rowy) * 10.0)

        def fused(i, carry):
            ps, rn, mx, am = carry
            sl = pl.ds(i * CH, CH)
            q = (jnp.square(sex[sl, :] - cx) * ssx
                 + jnp.square(sey[sl, :] - cy) * ssy)
            prop = q < _LN2
            prop8[sl, :] = prop.astype(jnp.int8)
            sc = score[sl, :]
            ps = ps + jnp.sum(prop.astype(f32))
            rn = rn + jnp.sum((prop & (sc > 0)).astype(f32))
            sc = jnp.where(prop, f32(0.0), sc)
            score[sl, :] = sc
            m = jnp.max(sc)
            cand = jnp.min(jnp.where(sc == m, chunk_iota(i, CH), i32(_BIG)))
            take = m > mx
            return (ps, rn, jnp.where(take, m, mx),
                    jnp.where(take, cand, am))

        psum, rnum, mx, am = jax.lax.fori_loop(
            0, N_CHUNKS, fused,
            (f32(0.0), f32(0.0), f32(-1.0), i32(0)))
        tot = tot - rnum
        rnum = rnum - 1.0
        add = (psum > 160.0) & (rnum / jnp.maximum(psum, 1.0) > 0.5)

        @pl.when(add)
        def _():
            def assign(i, _):
                sl = pl.ds(i * CH, CH)
                pr = prop8[sl, :].astype(i32) != 0
                lab[sl, :] = jnp.where(pr, cnt, lab[sl, :])
                return 0

            jax.lax.fori_loop(0, N_CHUNKS, assign, 0)

        cnt = cnt + add.astype(i32)
        return (tot, mx, am, cnt)

    jax.lax.while_loop(cond_fn, body_fn, (tot0, mx0, am0, jnp.int32(1)))

    def emit(i, _):
        sl = pl.ds(i * CH, CH)
        l = jnp.maximum(lab[sl, :], 0)
        out[sl, :] = jnp.bitwise_and(l, 255).astype(jnp.uint8)
        return 0

    jax.lax.fori_loop(0, N_CHUNKS, emit, 0)


def kernel(prediction, xym):
    pred = prediction[0]
    H, W = pred.shape[1], pred.shape[2]
    CHP = H // N_PREP
    xx = xym[0, 0:H, 0:W]
    yy = xym[1, 0:H, 0:W]

    hbm = pl.BlockSpec(memory_space=pl.ANY)
    inst = pl.pallas_call(
        _cluster_body,
        in_specs=[hbm] * 7,
        out_specs=pl.BlockSpec(memory_space=pltpu.VMEM),
        out_shape=jax.ShapeDtypeStruct((H, W), jnp.uint8),
        scratch_shapes=[
            pltpu.VMEM((H, W), jnp.float32),
            pltpu.VMEM((H, W), jnp.float32),
            pltpu.VMEM((H, W), jnp.float32),
            pltpu.VMEM((H, W), jnp.int32),
            pltpu.VMEM((H, W), jnp.int8),
            pltpu.VMEM((2, 5, CHP, W), jnp.float32),
            pltpu.VMEM((1, W), jnp.float32),
            pltpu.VMEM((1, W), jnp.float32),
            pltpu.SemaphoreType.DMA((2, 5)),
            pltpu.SemaphoreType.DMA((2,)),
        ],
    )(pred[0], pred[1], pred[6], xx, yy, pred[2], pred[3])

    return inst[None]


# speedup vs baseline: 51.4132x; 1.0166x over previous
import jax
import jax.numpy as jnp
from jax.experimental import pallas as pl
from jax.experimental.pallas import tpu as pltpu

N_CHUNKS = 2
N_PREP = 16
_LN2 = 0.6931471805599453
_BIG = 2147483647


def _cluster_body(p0, p1, p6, xx, yy, sgx, sgy, out,
                  sex, sey, score, lab, prop8, fidx, stage, rowx, rowy,
                  psem, rsem):
    H, W = sex.shape
    CH = H // N_CHUNKS
    CHP = H // N_PREP
    f32 = jnp.float32
    i32 = jnp.int32
    INF = jnp.float32(jnp.inf)
    srcs = (p0, p1, p6, xx, yy)

    def chunk_iota(i, rows):
        r = jax.lax.broadcasted_iota(i32, (rows, W), 0) + i * rows
        c = jax.lax.broadcasted_iota(i32, (rows, W), 1)
        return r * W + c

    def stage_copy(slot, i, j):
        return pltpu.make_async_copy(
            srcs[j].at[pl.ds(i * CHP, CHP), :], stage.at[slot, j],
            psem.at[slot, j])

    def issue(slot, i):
        for j in range(5):
            stage_copy(slot, i, j).start()

    def wait_all(slot, i):
        for j in range(5):
            stage_copy(slot, i, j).wait()

    issue(0, 0)

    def prep_it(i, carry):
        tot, mx, am = carry
        slot = jax.lax.rem(i, 2)

        @pl.when(i + 1 < N_PREP)
        def _():
            issue(jax.lax.rem(i + 1, 2), i + 1)

        wait_all(slot, i)
        sl = pl.ds(i * CHP, CHP)
        fi = chunk_iota(i, CHP)
        fidx[sl, :] = fi
        s = jax.nn.sigmoid(stage[slot, 2])
        v = s > 0.5
        sex[sl, :] = jnp.where(v, jnp.tanh(stage[slot, 0]) + stage[slot, 3],
                               INF)
        sey[sl, :] = jnp.where(v, jnp.tanh(stage[slot, 1]) + stage[slot, 4],
                               INF)
        sc = jnp.where(v, s, f32(0.0))
        score[sl, :] = sc
        lab[sl, :] = jnp.where(v, i32(0), i32(-1))
        tot = tot + jnp.sum(v.astype(f32))
        m = jnp.max(sc)
        cand = jnp.min(jnp.where(sc == m, fi, i32(_BIG)))
        take = m > mx
        return (tot, jnp.where(take, m, mx), jnp.where(take, cand, am))

    tot0, mx0, am0 = jax.lax.fori_loop(
        0, N_PREP, prep_it, (f32(0.0), f32(-1.0), i32(0)))

    def cond_fn(st):
        tot, mx, am, cnt = st
        return (tot > 160.0) & (mx >= 0.5)

    def body_fn(st):
        tot, mx, am, cnt = st
        r = am // W
        c = am % W
        cpx = pltpu.make_async_copy(sgx.at[pl.ds(r, 1), :], rowx,
                                    rsem.at[0])
        cpy = pltpu.make_async_copy(sgy.at[pl.ds(r, 1), :], rowy,
                                    rsem.at[1])
        cpx.start()
        cpy.start()
        sel = jax.lax.broadcasted_iota(i32, (1, W), 1) == c

        def pick(ref):
            return jnp.sum(jnp.where(sel, ref[pl.ds(r, 1), :], f32(0.0)))

        def pick_row(ref):
            return jnp.sum(jnp.where(sel, ref[...], f32(0.0)))

        cx = pick(sex)
        cy = pick(sey)
        cpx.wait()
        cpy.wait()
        ssx = jnp.exp(pick_row(rowx) * 10.0)
        ssy = jnp.exp(pick_row(rowy) * 10.0)

        def fused(i, carry):
            ps, rn, mx, am = carry
            sl = pl.ds(i * CH, CH)
            q = (jnp.square(sex[sl, :] - cx) * ssx
                 + jnp.square(sey[sl, :] - cy) * ssy)
            prop = q < _LN2
            prop8[sl, :] = prop.astype(jnp.int8)
            sc = score[sl, :]
            ps = ps + jnp.sum(prop.astype(f32))
            rn = rn + jnp.sum((prop & (sc > 0)).astype(f32))
            sc = jnp.where(prop, f32(0.0), sc)
            score[sl, :] = sc
            m = jnp.max(sc)
            cand = jnp.min(jnp.where(sc == m, fidx[sl, :], i32(_BIG)))
            take = m > mx
            return (ps, rn, jnp.where(take, m, mx),
                    jnp.where(take, cand, am))

        psum, rnum, mx, am = jax.lax.fori_loop(
            0, N_CHUNKS, fused,
            (f32(0.0), f32(0.0), f32(-1.0), i32(0)))
        tot = tot - rnum
        rnum = rnum - 1.0
        add = (psum > 160.0) & (rnum / jnp.maximum(psum, 1.0) > 0.5)

        @pl.when(add)
        def _():
            def assign(i, _):
                sl = pl.ds(i * CH, CH)
                pr = prop8[sl, :].astype(i32) != 0
                lab[sl, :] = jnp.where(pr, cnt, lab[sl, :])
                return 0

            jax.lax.fori_loop(0, N_CHUNKS, assign, 0)

        cnt = cnt + add.astype(i32)
        return (tot, mx, am, cnt)

    jax.lax.while_loop(cond_fn, body_fn, (tot0, mx0, am0, jnp.int32(1)))

    def emit(i, _):
        sl = pl.ds(i * CH, CH)
        l = jnp.maximum(lab[sl, :], 0)
        out[sl, :] = jnp.bitwise_and(l, 255).astype(jnp.uint8)
        return 0

    jax.lax.fori_loop(0, N_CHUNKS, emit, 0)


def kernel(prediction, xym):
    pred = prediction[0]
    H, W = pred.shape[1], pred.shape[2]
    CHP = H // N_PREP
    xx = xym[0, 0:H, 0:W]
    yy = xym[1, 0:H, 0:W]

    hbm = pl.BlockSpec(memory_space=pl.ANY)
    inst = pl.pallas_call(
        _cluster_body,
        in_specs=[hbm] * 7,
        out_specs=pl.BlockSpec(memory_space=pltpu.VMEM),
        out_shape=jax.ShapeDtypeStruct((H, W), jnp.uint8),
        scratch_shapes=[
            pltpu.VMEM((H, W), jnp.float32),
            pltpu.VMEM((H, W), jnp.float32),
            pltpu.VMEM((H, W), jnp.float32),
            pltpu.VMEM((H, W), jnp.int32),
            pltpu.VMEM((H, W), jnp.int8),
            pltpu.VMEM((H, W), jnp.int32),
            pltpu.VMEM((2, 5, CHP, W), jnp.float32),
            pltpu.VMEM((1, W), jnp.float32),
            pltpu.VMEM((1, W), jnp.float32),
            pltpu.SemaphoreType.DMA((2, 5)),
            pltpu.SemaphoreType.DMA((2,)),
        ],
    )(pred[0], pred[1], pred[6], xx, yy, pred[2], pred[3])

    return inst[None]


# speedup vs baseline: 56.6527x; 1.1019x over previous
import jax
import jax.numpy as jnp
from jax.experimental import pallas as pl
from jax.experimental.pallas import tpu as pltpu

N_CHUNKS = 2
N_PREP = 16
_LN2 = 0.6931471805599453
_BIG = 2147483647


def _cluster_body(p0, p1, p6, xrow, ycol, sgx, sgy, out,
                  sex, sey, score, lab, prop8, fidx, stage, rowx, rowy,
                  psem, rsem):
    H, W = sex.shape
    CH = H // N_CHUNKS
    CHP = H // N_PREP
    f32 = jnp.float32
    i32 = jnp.int32
    INF = jnp.float32(jnp.inf)
    srcs = (p0, p1, p6)

    def chunk_iota(i, rows):
        r = jax.lax.broadcasted_iota(i32, (rows, W), 0) + i * rows
        c = jax.lax.broadcasted_iota(i32, (rows, W), 1)
        return r * W + c

    def stage_copy(slot, i, j):
        return pltpu.make_async_copy(
            srcs[j].at[pl.ds(i * CHP, CHP), :], stage.at[slot, j],
            psem.at[slot, j])

    def issue(slot, i):
        for j in range(3):
            stage_copy(slot, i, j).start()

    def wait_all(slot, i):
        for j in range(3):
            stage_copy(slot, i, j).wait()

    issue(0, 0)

    def prep_it(i, carry):
        tot, mx, am = carry
        slot = jax.lax.rem(i, 2)

        @pl.when(i + 1 < N_PREP)
        def _():
            issue(jax.lax.rem(i + 1, 2), i + 1)

        wait_all(slot, i)
        sl = pl.ds(i * CHP, CHP)
        fi = chunk_iota(i, CHP)
        fidx[sl, :] = fi
        s = jax.nn.sigmoid(stage[slot, 2])
        v = s > 0.5
        sex[sl, :] = jnp.where(v, jnp.tanh(stage[slot, 0]) + xrow[...], INF)
        sey[sl, :] = jnp.where(v, jnp.tanh(stage[slot, 1]) + ycol[sl, :], INF)
        sc = jnp.where(v, s, f32(0.0))
        score[sl, :] = sc
        lab[sl, :] = jnp.where(v, i32(0), i32(-1))
        tot = tot + jnp.sum(v.astype(f32))
        m = jnp.max(sc)
        cand = jnp.min(jnp.where(sc == m, fi, i32(_BIG)))
        take = m > mx
        return (tot, jnp.where(take, m, mx), jnp.where(take, cand, am))

    tot0, mx0, am0 = jax.lax.fori_loop(
        0, N_PREP, prep_it, (f32(0.0), f32(-1.0), i32(0)))

    def cond_fn(st):
        tot, mx, am, cnt = st
        return (tot > 160.0) & (mx >= 0.5)

    def body_fn(st):
        tot, mx, am, cnt = st
        r = am // W
        c = am % W
        cpx = pltpu.make_async_copy(sgx.at[pl.ds(r, 1), :], rowx,
                                    rsem.at[0])
        cpy = pltpu.make_async_copy(sgy.at[pl.ds(r, 1), :], rowy,
                                    rsem.at[1])
        cpx.start()
        cpy.start()
        sel = jax.lax.broadcasted_iota(i32, (1, W), 1) == c

        def pick(ref):
            return jnp.sum(jnp.where(sel, ref[pl.ds(r, 1), :], f32(0.0)))

        def pick_row(ref):
            return jnp.sum(jnp.where(sel, ref[...], f32(0.0)))

        cx = pick(sex)
        cy = pick(sey)
        cpx.wait()
        cpy.wait()
        ssx = jnp.exp(pick_row(rowx) * 10.0)
        ssy = jnp.exp(pick_row(rowy) * 10.0)

        def fused(i, carry):
            ps, rn, mx, am = carry
            sl = pl.ds(i * CH, CH)
            q = (jnp.square(sex[sl, :] - cx) * ssx
                 + jnp.square(sey[sl, :] - cy) * ssy)
            prop = q < _LN2
            prop8[sl, :] = prop.astype(jnp.int8)
            sc = score[sl, :]
            ps = ps + jnp.sum(prop.astype(f32))
            rn = rn + jnp.sum((prop & (sc > 0)).astype(f32))
            sc = jnp.where(prop, f32(0.0), sc)
            score[sl, :] = sc
            m = jnp.max(sc)
            cand = jnp.min(jnp.where(sc == m, fidx[sl, :], i32(_BIG)))
            take = m > mx
            return (ps, rn, jnp.where(take, m, mx),
                    jnp.where(take, cand, am))

        psum, rnum, mx, am = jax.lax.fori_loop(
            0, N_CHUNKS, fused,
            (f32(0.0), f32(0.0), f32(-1.0), i32(0)))
        tot = tot - rnum
        rnum = rnum - 1.0
        add = (psum > 160.0) & (rnum / jnp.maximum(psum, 1.0) > 0.5)

        @pl.when(add)
        def _():
            def assign(i, _):
                sl = pl.ds(i * CH, CH)
                pr = prop8[sl, :].astype(i32) != 0
                lab[sl, :] = jnp.where(pr, cnt, lab[sl, :])
                return 0

            jax.lax.fori_loop(0, N_CHUNKS, assign, 0)

        cnt = cnt + add.astype(i32)
        return (tot, mx, am, cnt)

    jax.lax.while_loop(cond_fn, body_fn, (tot0, mx0, am0, jnp.int32(1)))

    def emit(i, _):
        sl = pl.ds(i * CH, CH)
        l = jnp.maximum(lab[sl, :], 0)
        out[sl, :] = jnp.bitwise_and(l, 255).astype(jnp.uint8)
        return 0

    jax.lax.fori_loop(0, N_CHUNKS, emit, 0)


def kernel(prediction, xym):
    pred = prediction[0]
    H, W = pred.shape[1], pred.shape[2]
    CHP = H // N_PREP
    xrow = xym[0, 0:1, 0:W]
    ycol = xym[1, 0:H, 0:1]

    hbm = pl.BlockSpec(memory_space=pl.ANY)
    vmem = pl.BlockSpec(memory_space=pltpu.VMEM)
    inst = pl.pallas_call(
        _cluster_body,
        in_specs=[hbm, hbm, hbm, vmem, vmem, hbm, hbm],
        out_specs=pl.BlockSpec(memory_space=pltpu.VMEM),
        out_shape=jax.ShapeDtypeStruct((H, W), jnp.uint8),
        scratch_shapes=[
            pltpu.VMEM((H, W), jnp.float32),
            pltpu.VMEM((H, W), jnp.float32),
            pltpu.VMEM((H, W), jnp.float32),
            pltpu.VMEM((H, W), jnp.int32),
            pltpu.VMEM((H, W), jnp.int8),
            pltpu.VMEM((H, W), jnp.int32),
            pltpu.VMEM((2, 3, CHP, W), jnp.float32),
            pltpu.VMEM((1, W), jnp.float32),
            pltpu.VMEM((1, W), jnp.float32),
            pltpu.SemaphoreType.DMA((2, 3)),
            pltpu.SemaphoreType.DMA((2,)),
        ],
    )(pred[0], pred[1], pred[6], xrow, ycol, pred[2], pred[3])

    return inst[None]


# speedup vs baseline: 59.1950x; 1.0449x over previous
import jax
import jax.numpy as jnp
from jax.experimental import pallas as pl
from jax.experimental.pallas import tpu as pltpu

N_CHUNKS = 2
N_PREP = 8
_LN2 = 0.6931471805599453
_BIG = 2147483647


def _cluster_body(p0, p1, p6, xrow, ycol, sgx, sgy, out,
                  sex, sey, score, lab, prop8, fidx, stage, rowx, rowy,
                  psem, rsem):
    H, W = sex.shape
    CH = H // N_CHUNKS
    CHP = H // N_PREP
    f32 = jnp.float32
    i32 = jnp.int32
    INF = jnp.float32(jnp.inf)
    srcs = (p0, p1, p6)

    def chunk_iota(i, rows):
        r = jax.lax.broadcasted_iota(i32, (rows, W), 0) + i * rows
        c = jax.lax.broadcasted_iota(i32, (rows, W), 1)
        return r * W + c

    def stage_copy(slot, i, j):
        return pltpu.make_async_copy(
            srcs[j].at[pl.ds(i * CHP, CHP), :], stage.at[slot, j],
            psem.at[slot, j])

    def issue(slot, i):
        for j in range(3):
            stage_copy(slot, i, j).start()

    def wait_all(slot, i):
        for j in range(3):
            stage_copy(slot, i, j).wait()

    issue(0, 0)

    def prep_it(i, carry):
        tot, mx, am = carry
        slot = jax.lax.rem(i, 2)

        @pl.when(i + 1 < N_PREP)
        def _():
            issue(jax.lax.rem(i + 1, 2), i + 1)

        wait_all(slot, i)
        sl = pl.ds(i * CHP, CHP)
        fi = chunk_iota(i, CHP)
        fidx[sl, :] = fi
        s = jax.nn.sigmoid(stage[slot, 2])
        v = s > 0.5
        sex[sl, :] = jnp.where(v, jnp.tanh(stage[slot, 0]) + xrow[...], INF)
        sey[sl, :] = jnp.where(v, jnp.tanh(stage[slot, 1]) + ycol[sl, :], INF)
        sc = jnp.where(v, s, f32(0.0))
        score[sl, :] = sc
        lab[sl, :] = jnp.where(v, i32(0), i32(-1))
        tot = tot + jnp.sum(v.astype(f32))
        m = jnp.max(sc)
        cand = jnp.min(jnp.where(sc == m, fi, i32(_BIG)))
        take = m > mx
        return (tot, jnp.where(take, m, mx), jnp.where(take, cand, am))

    tot0, mx0, am0 = jax.lax.fori_loop(
        0, N_PREP, prep_it, (f32(0.0), f32(-1.0), i32(0)))

    def cond_fn(st):
        tot, mx, am, cnt = st
        return (tot > 160.0) & (mx >= 0.5)

    def body_fn(st):
        tot, mx, am, cnt = st
        r = am // W
        c = am % W
        cpx = pltpu.make_async_copy(sgx.at[pl.ds(r, 1), :], rowx,
                                    rsem.at[0])
        cpy = pltpu.make_async_copy(sgy.at[pl.ds(r, 1), :], rowy,
                                    rsem.at[1])
        cpx.start()
        cpy.start()
        sel = jax.lax.broadcasted_iota(i32, (1, W), 1) == c

        def pick(ref):
            return jnp.sum(jnp.where(sel, ref[pl.ds(r, 1), :], f32(0.0)))

        def pick_row(ref):
            return jnp.sum(jnp.where(sel, ref[...], f32(0.0)))

        cx = pick(sex)
        cy = pick(sey)
        cpx.wait()
        cpy.wait()
        ssx = jnp.exp(pick_row(rowx) * 10.0)
        ssy = jnp.exp(pick_row(rowy) * 10.0)

        def fused(i, carry):
            ps, rn, mx, am = carry
            sl = pl.ds(i * CH, CH)
            q = (jnp.square(sex[sl, :] - cx) * ssx
                 + jnp.square(sey[sl, :] - cy) * ssy)
            prop = q < _LN2
            prop8[sl, :] = prop.astype(jnp.int8)
            sc = score[sl, :]
            ps = ps + jnp.sum(prop.astype(f32))
            rn = rn + jnp.sum((prop & (sc > 0)).astype(f32))
            sc = jnp.where(prop, f32(0.0), sc)
            score[sl, :] = sc
            m = jnp.max(sc)
            cand = jnp.min(jnp.where(sc == m, fidx[sl, :], i32(_BIG)))
            take = m > mx
            return (ps, rn, jnp.where(take, m, mx),
                    jnp.where(take, cand, am))

        psum, rnum, mx, am = jax.lax.fori_loop(
            0, N_CHUNKS, fused,
            (f32(0.0), f32(0.0), f32(-1.0), i32(0)))
        tot = tot - rnum
        rnum = rnum - 1.0
        add = (psum > 160.0) & (rnum / jnp.maximum(psum, 1.0) > 0.5)

        @pl.when(add)
        def _():
            def assign(i, _):
                sl = pl.ds(i * CH, CH)
                pr = prop8[sl, :].astype(i32) != 0
                lab[sl, :] = jnp.where(pr, cnt, lab[sl, :])
                return 0

            jax.lax.fori_loop(0, N_CHUNKS, assign, 0)

        cnt = cnt + add.astype(i32)
        return (tot, mx, am, cnt)

    jax.lax.while_loop(cond_fn, body_fn, (tot0, mx0, am0, jnp.int32(1)))

    def emit(i, _):
        sl = pl.ds(i * CH, CH)
        l = jnp.maximum(lab[sl, :], 0)
        out[sl, :] = jnp.bitwise_and(l, 255).astype(jnp.uint8)
        return 0

    jax.lax.fori_loop(0, N_CHUNKS, emit, 0)


def kernel(prediction, xym):
    pred = prediction[0]
    H, W = pred.shape[1], pred.shape[2]
    CHP = H // N_PREP
    xrow = xym[0, 0:1, 0:W]
    ycol = xym[1, 0:H, 0:1]

    hbm = pl.BlockSpec(memory_space=pl.ANY)
    vmem = pl.BlockSpec(memory_space=pltpu.VMEM)
    inst = pl.pallas_call(
        _cluster_body,
        in_specs=[hbm, hbm, hbm, vmem, vmem, hbm, hbm],
        out_specs=pl.BlockSpec(memory_space=pltpu.VMEM),
        out_shape=jax.ShapeDtypeStruct((H, W), jnp.uint8),
        scratch_shapes=[
            pltpu.VMEM((H, W), jnp.float32),
            pltpu.VMEM((H, W), jnp.float32),
            pltpu.VMEM((H, W), jnp.float32),
            pltpu.VMEM((H, W), jnp.int32),
            pltpu.VMEM((H, W), jnp.int8),
            pltpu.VMEM((H, W), jnp.int32),
            pltpu.VMEM((2, 3, CHP, W), jnp.float32),
            pltpu.VMEM((1, W), jnp.float32),
            pltpu.VMEM((1, W), jnp.float32),
            pltpu.SemaphoreType.DMA((2, 3)),
            pltpu.SemaphoreType.DMA((2,)),
        ],
    )(pred[0], pred[1], pred[6], xrow, ycol, pred[2], pred[3])

    return inst[None]
